# baseline (speedup 1.0000x reference)
import math

import jax
import jax.numpy as jnp
from jax import lax
from jax.experimental import pallas as pl
from jax.experimental.pallas import tpu as pltpu
from jax.experimental.pallas import tpu_sc as plsc

_V = 1_000_000
_D = 64
_TW = 128
_LANES = 16
_NC, _NS = 2, 16
_NW = _NC * _NS
_B = 4096 * 200
_PER_W = _B // _NW
_C = 128
_NCHUNK = _PER_W // _C
_NBUF = 4
_SCALE = math.sqrt(_D)


def _emb_body(tok_hbm, table_hbm, out_hbm, idx_v, in_v, out_v, *sems):
    sem_in = sems[:_NBUF]
    sem_out = sems[_NBUF:]
    table2d = table_hbm
    wid = lax.axis_index("s") * _NC + lax.axis_index("c")
    base = wid * _PER_W

    pltpu.sync_copy(tok_hbm.at[wid], idx_v)

    def gather(g, b):
        return pltpu.make_async_copy(
            table2d.at[idx_v.at[g]], in_v.at[b], sem_in[b])

    def scatter(g, b):
        row0 = base + g * _C
        return pltpu.make_async_copy(
            out_v.at[b], out_hbm.at[pl.ds(row0, _C)], sem_out[b])

    def scale(b):
        def row(i, carry):
            for j in range(_D // _LANES):
                sl = pl.ds(j * _LANES, _LANES)
                out_v[b, i, sl] = in_v[b, i, sl] * _SCALE
            return carry
        lax.fori_loop(0, _C, row, 0, unroll=4)

    for b in range(_NBUF):
        gather(b, b).start()

    for b in range(_NBUF):
        gather(b, b).wait()
        scale(b)
        scatter(b, b).start()
        gather(_NBUF + b, b).start()

    def outer(t, carry):
        for b in range(_NBUF):
            g = t * _NBUF + b
            gather(g, b).wait()
            scatter(g, b).wait()
            scale(b)
            scatter(g, b).start()
            gather(g + _NBUF, b).start()
        return carry

    lax.fori_loop(1, _NCHUNK // _NBUF - 1, outer, 0)

    t_last = _NCHUNK // _NBUF - 1
    for b in range(_NBUF):
        g = t_last * _NBUF + b
        gather(g, b).wait()
        scatter(g, b).wait()
        scale(b)
        scatter(g, b).start()

    for b in range(_NBUF):
        scatter(0, b).wait()


_emb_call = pl.kernel(
    _emb_body,
    mesh=plsc.VectorSubcoreMesh(core_axis_name="c", subcore_axis_name="s"),
    out_type=jax.ShapeDtypeStruct((_B, _D), jnp.float32),
    scratch_types=[
        pltpu.VMEM((_NCHUNK, _C), jnp.int32),
        pltpu.VMEM((_NBUF, _C, _TW), jnp.float32),
        pltpu.VMEM((_NBUF, _C, _D), jnp.float32),
    ] + [pltpu.SemaphoreType.DMA] * (2 * _NBUF),
    compiler_params=pltpu.CompilerParams(use_tc_tiling_on_sc=False),
)


def kernel(tokens, table):
    tok3 = tokens.reshape(_NW, _NCHUNK, _C).astype(jnp.int32)
    tpad = jnp.pad(table, ((0, 0), (0, _TW - _D)))
    out = _emb_call(tok3, tpad)
    return out.reshape(*tokens.shape, _D)
